# baseline (speedup 1.0000x reference)
import functools

import jax
import jax.numpy as jnp
from jax import lax
from jax.experimental import pallas as pl
from jax.experimental.pallas import tpu as pltpu
from jax.experimental.pallas import tpu_sc as plsc

_D = 32
_NC = 2
_NS = 16
_NW = _NC * _NS
_SEG = 128
_K = 16


def _sc_gather(idx2d, table):
    n_rows = idx2d.shape[0]
    n_chunks = n_rows // _K
    rounds = (n_chunks + _NW - 1) // _NW
    b_total = n_rows * _SEG

    @functools.partial(
        pl.kernel,
        out_type=jax.ShapeDtypeStruct((b_total, _D), jnp.float32),
        mesh=plsc.VectorSubcoreMesh(core_axis_name="c", subcore_axis_name="s"),
        compiler_params=pltpu.CompilerParams(use_tc_tiling_on_sc=False),
        scratch_types=[
            pltpu.VMEM((_K, _SEG), jnp.int32),
            pltpu.VMEM((_K, _SEG), jnp.int32),
            pltpu.VMEM((_K * _SEG, _D), jnp.float32),
            pltpu.SemaphoreType.DMA,
            pltpu.SemaphoreType.DMA,
            pltpu.SemaphoreType.DMA,
        ],
    )
    def k(idx_hbm, table_hbm, out_hbm, idx_v0, idx_v1, rows_v,
          gsem, isem0, isem1):
        wid = lax.axis_index("s") * _NC + lax.axis_index("c")
        idx_bufs = (idx_v0, idx_v1)
        isems = (isem0, isem1)

        def idx_copy(g, p):
            row0 = (g * _NW + wid) * _K
            return pltpu.make_async_copy(
                idx_hbm.at[pl.ds(row0, _K)], idx_bufs[p], isems[p])

        @pl.when(wid < n_chunks)
        def _():
            idx_copy(0, 0).start()

        def sub_iter(g, p):
            chunk = g * _NW + wid

            @pl.when(chunk < n_chunks)
            def _():
                idx_copy(g, p).wait()

                @pl.when((g + 1) * _NW + wid < n_chunks)
                def _():
                    idx_copy(g + 1, 1 - p).start()

                copies = []
                for j in range(_K):
                    copies.append(
                        pltpu.async_copy(
                            table_hbm.at[idx_bufs[p].at[j]],
                            rows_v.at[pl.ds(j * _SEG, _SEG)],
                            gsem,
                        )
                    )
                for c in copies:
                    c.wait()
                pltpu.sync_copy(
                    rows_v,
                    out_hbm.at[pl.ds(chunk * _K * _SEG, _K * _SEG)])

        def body(g2, carry):
            sub_iter(2 * g2, 0)
            sub_iter(2 * g2 + 1, 1)
            return carry

        lax.fori_loop(0, (rounds + 1) // 2, body, 0)

    return k(idx2d, table)


def kernel(input, weight):
    b, h = input.shape
    idx2d = input.reshape(b * h // _SEG, _SEG)
    out = _sc_gather(idx2d, weight)
    return out.reshape(b, h, _D)


# speedup vs baseline: 1.6227x; 1.6227x over previous
import functools

import jax
import jax.numpy as jnp
from jax import lax
from jax.experimental import pallas as pl
from jax.experimental.pallas import tpu as pltpu
from jax.experimental.pallas import tpu_sc as plsc

_D = 32
_NC = 2
_NS = 16
_NW = _NC * _NS
_CB = 32


def _sc_gather(idx, table):
    batch, hist = idx.shape
    per_w = batch // _NW
    n_chunks = per_w // _CB

    @functools.partial(
        pl.kernel,
        out_type=jax.ShapeDtypeStruct((batch, hist, _D), jnp.float32),
        mesh=plsc.VectorSubcoreMesh(core_axis_name="c", subcore_axis_name="s"),
        compiler_params=pltpu.CompilerParams(use_tc_tiling_on_sc=False),
        scratch_types=[
            pltpu.VMEM((_CB, hist), jnp.int32),
            pltpu.VMEM((_CB, hist), jnp.int32),
            pltpu.VMEM((_CB, hist, _D), jnp.float32),
            pltpu.VMEM((_CB, hist, _D), jnp.float32),
            pltpu.SemaphoreType.DMA,
            pltpu.SemaphoreType.DMA,
            pltpu.SemaphoreType.DMA,
            pltpu.SemaphoreType.DMA,
            pltpu.SemaphoreType.DMA,
        ],
    )
    def k(idx_hbm, table_hbm, out_hbm, idx_v0, idx_v1, rows_v0, rows_v1,
          gsem, isem0, isem1, osem0, osem1):
        wid = lax.axis_index("s") * _NC + lax.axis_index("c")
        base = wid * per_w
        idx_bufs = (idx_v0, idx_v1)
        rows_bufs = (rows_v0, rows_v1)
        isems = (isem0, isem1)
        osems = (osem0, osem1)

        def idx_copy(g, p):
            return pltpu.make_async_copy(
                idx_hbm.at[pl.ds(base + g * _CB, _CB)], idx_bufs[p], isems[p])

        def out_copy(g, p):
            return pltpu.make_async_copy(
                rows_bufs[p], out_hbm.at[pl.ds(base + g * _CB, _CB)],
                osems[p])

        idx_copy(0, 0).start()

        def sub_iter(g, p):
            idx_copy(g, p).wait()

            @pl.when(g + 1 < n_chunks)
            def _():
                idx_copy(g + 1, 1 - p).start()

            @pl.when(g >= 2)
            def _():
                out_copy(g - 2, p).wait()

            def fire(b, carry):
                pltpu.make_async_copy(
                    table_hbm.at[idx_bufs[p].at[b]],
                    rows_bufs[p].at[b],
                    gsem,
                ).start()
                return carry

            lax.fori_loop(0, _CB, fire, 0)
            pltpu.make_async_copy(
                out_hbm.at[pl.ds(base + g * _CB, _CB)], rows_bufs[p],
                gsem).wait()
            out_copy(g, p).start()

        def body(g2, carry):
            sub_iter(2 * g2, 0)
            sub_iter(2 * g2 + 1, 1)
            return carry

        lax.fori_loop(0, n_chunks // 2, body, 0)
        out_copy(n_chunks - 2, (n_chunks - 2) % 2).wait()
        out_copy(n_chunks - 1, (n_chunks - 1) % 2).wait()

    return k(idx, table)


def kernel(input, weight):
    return _sc_gather(input, weight)
